# baseline (speedup 1.0000x reference)
import jax
import jax.numpy as jnp
from jax.experimental import pallas as pl
from jax.experimental.pallas import tpu as pltpu

_T, _A, _E, _D, _F = 8, 2, 8, 1024, 2816
_FT = 1408
_NF = _F // _FT
_NS = _NF + 1


def _ffn_kernel(idx_ref, meta_ref, x_ref, w1_ref, w2_ref, w3_ref, out_ref,
                h_ref):
    e = pl.program_id(0)
    s = pl.program_id(1)
    expert = meta_ref[e]
    valid = e < meta_ref[_E]
    dims = (((1,), (1,)), ((), ()))

    for fs in range(_NF):
        @pl.when(valid & (s == fs))
        def _h_tile():
            xb = x_ref[...]
            x1 = jax.lax.dot_general(xb, w1_ref[0], dims,
                                     preferred_element_type=jnp.float32)
            x3 = jax.lax.dot_general(xb, w3_ref[0], dims,
                                     preferred_element_type=jnp.float32)
            h_ref[:, fs * _FT:(fs + 1) * _FT] = (x1 * jax.nn.sigmoid(x1)) * x3

    @pl.when(valid & (s == _NF))
    def _contract():
        res = jax.lax.dot_general(h_ref[...], w2_ref[0], dims,
                                  preferred_element_type=jnp.float32)
        for p in range(_T * _A):
            @pl.when(idx_ref[p] == expert)
            def _write():
                out_ref[p, :] = res[p // _A, :]


def kernel(x, expert_indices, w1, w2, w3):
    idx = expert_indices.reshape(-1).astype(jnp.int32)
    present = jnp.zeros((_E,), jnp.bool_).at[idx].set(True)
    n_used = jnp.sum(present.astype(jnp.int32))
    order = jnp.argsort(jnp.logical_not(present)).astype(jnp.int32)
    sched = order[jnp.minimum(jnp.arange(_E), n_used - 1)]
    meta = jnp.concatenate([sched, n_used[None]])

    def _w13(e, s, idx_ref, m):
        live = (e < m[_E]) & (s < _NF)
        return (m[e], jnp.where(live, s, _NF - 1), 0)

    def _w2(e, s, idx_ref, m):
        return (m[e], 0, 0)

    grid_spec = pltpu.PrefetchScalarGridSpec(
        num_scalar_prefetch=2,
        grid=(_E, _NS),
        in_specs=[
            pl.BlockSpec((_T, _D), lambda e, s, i, m: (0, 0)),
            pl.BlockSpec((1, _FT, _D), _w13),
            pl.BlockSpec((1, _D, _F), _w2),
            pl.BlockSpec((1, _FT, _D), _w13),
        ],
        out_specs=pl.BlockSpec((_T * _A, _D), lambda e, s, i, m: (0, 0)),
        scratch_shapes=[pltpu.VMEM((_T, _F), jnp.float32)],
    )
    out = pl.pallas_call(
        _ffn_kernel,
        grid_spec=grid_spec,
        out_shape=jax.ShapeDtypeStruct((_T * _A, _D), jnp.float32),
    )(idx, meta, x, w1, w2, w3)
    return out.reshape(_T, _A, _D)
